# baseline (speedup 1.0000x reference)
import functools

import jax
import jax.numpy as jnp
from jax import lax
from jax.experimental import pallas as pl
from jax.experimental.pallas import tpu as pltpu
from jax.experimental.pallas import tpu_sc as plsc

_LANES = 16


@functools.cache
def _gather_rows_tiled_fn(B, V, D, num_cores, num_subcores):
    nw = num_cores * num_subcores
    assert B % (8 * nw) == 0
    b_per_w = B // nw
    chunk = 16
    passes = 4
    rows_per_pass = b_per_w // passes
    chunks_per_pass = rows_per_pass // chunk
    mesh = plsc.VectorSubcoreMesh(core_axis_name="c", subcore_axis_name="s")

    @functools.partial(
        pl.kernel,
        mesh=mesh,
        out_type=jax.ShapeDtypeStruct((B, D), jnp.float32),
        scratch_types=[
            pltpu.VMEM((b_per_w,), jnp.int32),
            pltpu.VMEM((rows_per_pass, D), jnp.float32),
            pltpu.SemaphoreType.DMA,
        ],
    )
    def k(idx_hbm, tab_hbm, out_hbm, idx_v, rows_v, sem):
        wid = lax.axis_index("s") * num_cores + lax.axis_index("c")
        base = wid * b_per_w
        pltpu.sync_copy(idx_hbm.at[pl.ds(base, b_per_w)], idx_v)

        for p in range(passes):
            pbase = p * rows_per_pass

            def chunk_body(g, carry, pbase=pbase):
                r0 = pbase + g * chunk
                rb = g * chunk
                vec = idx_v[pl.ds(r0, chunk)]
                copies = []
                for j in range(chunk):
                    copies.append(pltpu.async_copy(
                        tab_hbm.at[vec[j]], rows_v.at[rb + j], sem))
                for c in copies:
                    c.wait()
                return carry

            lax.fori_loop(0, chunks_per_pass, chunk_body, 0)
            pltpu.sync_copy(rows_v, out_hbm.at[pl.ds(base + pbase, rows_per_pass)])

    return k


@functools.cache
def _gather_rows_linear_fn(B, V, D, num_cores, num_subcores):
    nw = num_cores * num_subcores
    assert B % (8 * nw) == 0
    b_per_w = B // nw
    mesh = plsc.VectorSubcoreMesh(core_axis_name="c", subcore_axis_name="s")

    @functools.partial(
        pl.kernel,
        mesh=mesh,
        compiler_params=pltpu.CompilerParams(use_tc_tiling_on_sc=False),
        out_type=jax.ShapeDtypeStruct((B, D), jnp.float32),
        scratch_types=[
            pltpu.VMEM((b_per_w,), jnp.int32),
            pltpu.VMEM((b_per_w, D), jnp.float32),
            pltpu.SemaphoreType.DMA,
        ],
    )
    def k(idx_hbm, tab_hbm, out_hbm, idx_v, rows_v, sem):
        wid = lax.axis_index("s") * num_cores + lax.axis_index("c")
        base = wid * b_per_w
        pltpu.sync_copy(idx_hbm.at[pl.ds(base, b_per_w)], idx_v)
        pltpu.async_copy(tab_hbm.at[idx_v], rows_v, sem).wait()
        pltpu.sync_copy(rows_v, out_hbm.at[pl.ds(base, b_per_w)])

    return k


def _mlp_body(u_ref, i_ref, w1_ref, b1_ref, w2_ref, b2_ref, w3_ref, b3_ref,
              wm_ref, bm_ref, out_ref):
    x = u_ref[...] * i_ref[...]
    h = jnp.maximum(jnp.dot(x, w1_ref[...],
                            preferred_element_type=jnp.float32) + b1_ref[...], 0.0)
    h = jnp.maximum(jnp.dot(h, w2_ref[...],
                            preferred_element_type=jnp.float32) + b2_ref[...], 0.0)
    h = jnp.maximum(jnp.dot(h, w3_ref[...],
                            preferred_element_type=jnp.float32) + b3_ref[...], 0.0)
    o = jnp.dot(h, wm_ref[...], preferred_element_type=jnp.float32) + bm_ref[...]
    out_ref[...] = jax.nn.sigmoid(o[:, 0])


@functools.cache
def _mlp_fn(B, D, blk):
    grid = (B // blk,)
    full = lambda i: (0, 0)
    return pl.pallas_call(
        _mlp_body,
        grid=grid,
        in_specs=[
            pl.BlockSpec((blk, D), lambda i: (i, 0)),
            pl.BlockSpec((blk, D), lambda i: (i, 0)),
            pl.BlockSpec((64, 64), full),
            pl.BlockSpec((1, 64), full),
            pl.BlockSpec((64, 32), full),
            pl.BlockSpec((1, 32), full),
            pl.BlockSpec((32, 16), full),
            pl.BlockSpec((1, 16), full),
            pl.BlockSpec((16, 1), full),
            pl.BlockSpec((1, 1), full),
        ],
        out_specs=pl.BlockSpec((blk,), lambda i: (i,)),
        out_shape=jax.ShapeDtypeStruct((B,), jnp.float32),
    )


def kernel(user_indices, item_indices, user_table, item_table,
           W1, b1, W2, b2, W3, b3, Wm, bm):
    B = user_indices.shape[0]
    V, D = user_table.shape
    info = plsc.get_sparse_core_info()
    nc, ns = info.num_cores, info.num_subcores
    ue = _gather_rows_tiled_fn(B, V, D, nc, ns)(
        user_indices.astype(jnp.int32), user_table)
    ie = _gather_rows_linear_fn(B, item_table.shape[0], D, nc, ns)(
        item_indices.astype(jnp.int32), item_table)
    out = _mlp_fn(B, D, 1024)(
        ue, ie, W1, b1.reshape(1, -1), W2, b2.reshape(1, -1),
        W3, b3.reshape(1, -1), Wm, bm.reshape(1, -1))
    return out


# speedup vs baseline: 1.7761x; 1.7761x over previous
import functools

import jax
import jax.numpy as jnp
from jax import lax
from jax.experimental import pallas as pl
from jax.experimental.pallas import tpu as pltpu
from jax.experimental.pallas import tpu_sc as plsc

_LANES = 16


@functools.cache
def _gather_fn(B, D, num_cores, num_subcores, tab3d):
    nw = num_cores * num_subcores
    assert B % (8 * nw) == 0
    b_per_w = B // nw
    chunk = 16
    passes = 2
    rows_per_pass = b_per_w // passes
    chunks_per_pass = rows_per_pass // chunk
    mesh = plsc.VectorSubcoreMesh(core_axis_name="c", subcore_axis_name="s")

    @functools.partial(
        pl.kernel,
        mesh=mesh,
        out_type=jax.ShapeDtypeStruct((B, D), jnp.float32),
        scratch_types=[
            pltpu.VMEM((b_per_w,), jnp.int32),
            pltpu.VMEM((rows_per_pass, D), jnp.float32),
            pltpu.SemaphoreType.DMA,
        ],
    )
    def k(idx_hbm, tab_hbm, out_hbm, idx_v, rows_v, sem):
        wid = lax.axis_index("s") * num_cores + lax.axis_index("c")
        base = wid * b_per_w
        pltpu.sync_copy(idx_hbm.at[pl.ds(base, b_per_w)], idx_v)

        for p in range(passes):
            pbase = p * rows_per_pass

            def chunk_body(g, carry, pbase=pbase):
                r0 = pbase + g * chunk
                rb = g * chunk
                vec = idx_v[pl.ds(r0, chunk)]
                copies = []
                for j in range(chunk):
                    src = tab_hbm.at[0, vec[j]] if tab3d else tab_hbm.at[vec[j]]
                    copies.append(pltpu.async_copy(src, rows_v.at[rb + j], sem))
                for c in copies:
                    c.wait()
                return carry

            lax.fori_loop(0, chunks_per_pass, chunk_body, 0)
            pltpu.sync_copy(rows_v, out_hbm.at[pl.ds(base + pbase, rows_per_pass)])

    return k


def _mlp_body(u_ref, i_ref, w1_ref, b1_ref, w2_ref, b2_ref, w3_ref, b3_ref,
              wm_ref, bm_ref, out_ref):
    x = u_ref[...] * i_ref[...]
    h = jnp.maximum(jnp.dot(x, w1_ref[...],
                            preferred_element_type=jnp.float32) + b1_ref[...], 0.0)
    h = jnp.maximum(jnp.dot(h, w2_ref[...],
                            preferred_element_type=jnp.float32) + b2_ref[...], 0.0)
    h = jnp.maximum(jnp.dot(h, w3_ref[...],
                            preferred_element_type=jnp.float32) + b3_ref[...], 0.0)
    o = jnp.dot(h, wm_ref[...], preferred_element_type=jnp.float32) + bm_ref[...]
    out_ref[...] = jax.nn.sigmoid(o[:, 0])


@functools.cache
def _mlp_fn(B, D, blk):
    grid = (B // blk,)
    full = lambda i: (0, 0)
    return pl.pallas_call(
        _mlp_body,
        grid=grid,
        in_specs=[
            pl.BlockSpec((blk, D), lambda i: (i, 0)),
            pl.BlockSpec((blk, D), lambda i: (i, 0)),
            pl.BlockSpec((64, 64), full),
            pl.BlockSpec((1, 64), full),
            pl.BlockSpec((64, 32), full),
            pl.BlockSpec((1, 32), full),
            pl.BlockSpec((32, 16), full),
            pl.BlockSpec((1, 16), full),
            pl.BlockSpec((16, 1), full),
            pl.BlockSpec((1, 1), full),
        ],
        out_specs=pl.BlockSpec((blk,), lambda i: (i,)),
        out_shape=jax.ShapeDtypeStruct((B,), jnp.float32),
    )


def kernel(user_indices, item_indices, user_table, item_table,
           W1, b1, W2, b2, W3, b3, Wm, bm):
    B = user_indices.shape[0]
    V, D = user_table.shape
    info = plsc.get_sparse_core_info()
    nc, ns = info.num_cores, info.num_subcores
    ie = _gather_fn(B, D, nc, ns, True)(
        item_indices.astype(jnp.int32), item_table.reshape(1, V, D))
    ue = _gather_fn(B, D, nc, ns, False)(
        user_indices.astype(jnp.int32), user_table)
    out = _mlp_fn(B, D, 1024)(
        ue, ie, W1, b1.reshape(1, -1), W2, b2.reshape(1, -1),
        W3, b3.reshape(1, -1), Wm, bm.reshape(1, -1))
    return out
